# baseline (speedup 1.0000x reference)
import functools

import jax
import jax.numpy as jnp
from jax import lax
from jax.experimental import pallas as pl
from jax.experimental.pallas import tpu as pltpu
from jax.experimental.pallas import tpu_sc as plsc

_ANCHORS = [12.0, 16.0, 19.0, 36.0, 40.0, 28.0, 36.0, 75.0, 76.0, 55.0,
            72.0, 146.0, 142.0, 110.0, 192.0, 243.0, 459.0, 401.0]
_ANCHOR_MASK = [6, 7, 8]
_NCLS = 80
_STRIDE = 32
_CH = 256
_L = 16


def _sc_body(x_hbm, th_hbm, boxes_hbm, keep_hbm, t_v, ob_v, kb_v, th_s,
             *, B, A, H, W, aw, ah, nw):
    n = H * W
    cpb = n // _CH
    nchunks = B * cpb
    wid = lax.axis_index("s") * 2 + lax.axis_index("c")
    pltpu.sync_copy(th_hbm, th_s)
    th = th_s[:]
    iota = lax.iota(jnp.int32, _L)

    def chunk_body(it, carry):
        g = wid + it * nw
        b = g // cpb
        j0 = (g % cpb) * _CH
        pltpu.sync_copy(x_hbm.at[pl.ds(b, 1), :, :, pl.ds(j0, _CH)], t_v)
        for a in range(A):
            for v in range(_CH // _L):
                s16 = v * _L
                sl = pl.ds(s16, _L)
                ci = iota + (j0 + s16)
                gx = (ci & (W - 1)).astype(jnp.float32)
                gy = lax.shift_right_logical(
                    ci, W.bit_length() - 1).astype(jnp.float32)
                x0 = t_v[0, a, 0, sl]
                x1 = t_v[0, a, 1, sl]
                x2 = t_v[0, a, 2, sl]
                x3 = t_v[0, a, 3, sl]
                x4 = t_v[0, a, 4, sl]
                xs = (1.0 / (1.0 + jnp.exp(-x0)) + gx) * (1.0 / W)
                ys = (1.0 / (1.0 + jnp.exp(-x1)) + gy) * (1.0 / H)
                ws = jnp.exp(x2) * aw[a]
                hs = jnp.exp(x3) * ah[a]
                det = 1.0 / (1.0 + jnp.exp(-x4))

                def max_body(cc, mc):
                    vm, vi = mc
                    for k in range(8):
                        c = cc * 8 + k
                        cl = t_v[0, a, 5 + c, sl]
                        gt = cl > vm
                        cvec = jnp.full((_L,), 1, jnp.int32) * c
                        vm = jnp.where(gt, cl, vm)
                        vi = jnp.where(gt, cvec, vi)
                    return vm, vi

                vm0 = jnp.full((_L,), -jnp.inf, jnp.float32)
                vi0 = jnp.full((_L,), 0, jnp.int32)
                vm, vi = lax.fori_loop(0, _NCLS // 8, max_body, (vm0, vi0))

                def sum_body(cc, s):
                    for k in range(8):
                        c = cc * 8 + k
                        cl = t_v[0, a, 5 + c, sl]
                        s = s + jnp.exp(cl - vm)
                    return s

                s = lax.fori_loop(0, _NCLS // 8, sum_body,
                                  jnp.full((_L,), 0.0, jnp.float32))
                conf = 1.0 / s
                cid = vi.astype(jnp.float32)

                for f, val in enumerate([xs, ys, ws, hs, det, conf, cid]):
                    ob_v[0, 0, f, sl] = val
                kb_v[0, 0, sl] = jnp.where(det > th, 1.0, 0.0)
            pltpu.sync_copy(
                ob_v,
                boxes_hbm.at[pl.ds(b, 1), pl.ds(a, 1), :, pl.ds(j0, _CH)])
            pltpu.sync_copy(
                kb_v, keep_hbm.at[pl.ds(b, 1), pl.ds(a, 1), pl.ds(j0, _CH)])
        return carry

    nper = nchunks // nw
    lax.fori_loop(0, nper, chunk_body, 0)


def kernel(output, nms_tresh):
    B, C, H, W = output.shape
    A = len(_ANCHOR_MASK)
    n = H * W
    x = output.reshape(B, A, C // A, n)
    th = jnp.full((_L,), nms_tresh, jnp.float32)

    aw = tuple(_ANCHORS[m * 2] / _STRIDE / W for m in _ANCHOR_MASK)
    ah = tuple(_ANCHORS[m * 2 + 1] / _STRIDE / H for m in _ANCHOR_MASK)

    info = plsc.get_sparse_core_info()
    nw = info.num_cores * info.num_subcores
    mesh = plsc.VectorSubcoreMesh(core_axis_name="c", subcore_axis_name="s")

    body = functools.partial(_sc_body, B=B, A=A, H=H, W=W, aw=aw, ah=ah,
                             nw=nw)
    boxes_t, keepf = pl.kernel(
        body,
        mesh=mesh,
        out_type=[
            jax.ShapeDtypeStruct((B, A, 7, n), jnp.float32),
            jax.ShapeDtypeStruct((B, A, n), jnp.float32),
        ],
        scratch_types=[
            pltpu.VMEM((1, A, C // A, _CH), jnp.float32),
            pltpu.VMEM((1, 1, 7, _CH), jnp.float32),
            pltpu.VMEM((1, 1, _CH), jnp.float32),
            pltpu.VMEM((_L,), jnp.float32),
        ],
    )(x, th)

    boxes = boxes_t.transpose(0, 1, 3, 2).reshape(B, A * n, 7)
    keep = (keepf > 0.5).reshape(B, A * n)
    return boxes, keep


# speedup vs baseline: 1.2482x; 1.2482x over previous
import functools

import jax
import jax.numpy as jnp
from jax import lax
from jax.experimental import pallas as pl
from jax.experimental.pallas import tpu as pltpu
from jax.experimental.pallas import tpu_sc as plsc

_ANCHORS = [12.0, 16.0, 19.0, 36.0, 40.0, 28.0, 36.0, 75.0, 76.0, 55.0,
            72.0, 146.0, 142.0, 110.0, 192.0, 243.0, 459.0, 401.0]
_ANCHOR_MASK = [6, 7, 8]
_NCLS = 80
_STRIDE = 32
_BB = 8
_KSC = 24
_CH = 256
_L = 16


def _sigmoid(x):
    return 1.0 / (1.0 + jnp.exp(-x))


def _tc_body(thresh_ref, x_ref, boxes_ref, keep_ref, *, H, W, aw, ah, bb):
    n = H * W
    A = len(aw)
    idx = jax.lax.broadcasted_iota(jnp.int32, (1, n), 1)
    gx = (idx % W).astype(jnp.float32)
    gy = (idx // W).astype(jnp.float32)
    th = thresh_ref[0]

    for i in range(bb):
        for a in range(A):
            t = x_ref[i, a]
            xs = (_sigmoid(t[0:1, :]) + gx) * (1.0 / W)
            ys = (_sigmoid(t[1:2, :]) + gy) * (1.0 / H)
            ws = jnp.exp(t[2:3, :]) * aw[a]
            hs = jnp.exp(t[3:4, :]) * ah[a]
            det = _sigmoid(t[4:5, :])

            cls = t[5:5 + _NCLS, :]
            m = jnp.max(cls, axis=0, keepdims=True)
            s = jnp.sum(jnp.exp(cls - m), axis=0, keepdims=True)
            conf = 1.0 / s
            cidx = jax.lax.broadcasted_iota(jnp.int32, cls.shape, 0)
            first_max = jnp.min(
                jnp.where(cls == m, cidx, _NCLS), axis=0, keepdims=True)
            cid = first_max.astype(jnp.float32)

            boxes_ref[i, a] = jnp.concatenate(
                [xs, ys, ws, hs, det, conf, cid], axis=0)
            keep_ref[i, a] = det > th


def _tc_decode(x, th, Btc, *, A, C, H, W, aw, ah):
    n = H * W
    bb = _BB if Btc % _BB == 0 else 1
    body = functools.partial(_tc_body, H=H, W=W, aw=aw, ah=ah, bb=bb)
    return pl.pallas_call(
        body,
        grid=(Btc // bb,),
        in_specs=[
            pl.BlockSpec(memory_space=pltpu.SMEM),
            pl.BlockSpec((bb, A, C // A, n), lambda b: (b, 0, 0, 0)),
        ],
        out_specs=[
            pl.BlockSpec((bb, A, 7, n), lambda b: (b, 0, 0, 0)),
            pl.BlockSpec((bb, A, 1, n), lambda b: (b, 0, 0, 0)),
        ],
        out_shape=[
            jax.ShapeDtypeStruct((Btc, A, 7, n), jnp.float32),
            jax.ShapeDtypeStruct((Btc, A, 1, n), jnp.bool_),
        ],
    )(th.reshape(1), x)


def _sc_body(x_hbm, th_hbm, boxes_hbm, keep_hbm, t_v, ob_v, kb_v, th_v,
             *, A, H, W, aw, ah, nw, b_off, ksc):
    n = H * W
    cpb = n // _CH
    nchunks = ksc * cpb
    wid = lax.axis_index("s") * 2 + lax.axis_index("c")
    pltpu.sync_copy(th_hbm, th_v)
    th = th_v[:]
    iota = lax.iota(jnp.int32, _L)

    def chunk_body(it, carry):
        g = wid + it * nw
        b = g // cpb
        j0 = (g % cpb) * _CH
        pltpu.sync_copy(
            x_hbm.at[pl.ds(b_off + b, 1), :, :, pl.ds(j0, _CH)], t_v)
        for a in range(A):
            for v in range(_CH // _L):
                s16 = v * _L
                sl = pl.ds(s16, _L)
                ci = iota + (j0 + s16)
                gx = (ci & (W - 1)).astype(jnp.float32)
                gy = lax.shift_right_logical(
                    ci, W.bit_length() - 1).astype(jnp.float32)
                x0 = t_v[0, a, 0, sl]
                x1 = t_v[0, a, 1, sl]
                x2 = t_v[0, a, 2, sl]
                x3 = t_v[0, a, 3, sl]
                x4 = t_v[0, a, 4, sl]
                xs = (_sigmoid(x0) + gx) * (1.0 / W)
                ys = (_sigmoid(x1) + gy) * (1.0 / H)
                ws = jnp.exp(x2) * aw[a]
                hs = jnp.exp(x3) * ah[a]
                det = _sigmoid(x4)

                def max_body(cc, mc):
                    vm, vi = mc
                    for k in range(8):
                        c = cc * 8 + k
                        cl = t_v[0, a, 5 + c, sl]
                        gt = cl > vm
                        cvec = jnp.full((_L,), 1, jnp.int32) * c
                        vm = jnp.where(gt, cl, vm)
                        vi = jnp.where(gt, cvec, vi)
                    return vm, vi

                vm0 = jnp.full((_L,), -jnp.inf, jnp.float32)
                vi0 = jnp.full((_L,), 0, jnp.int32)
                vm, vi = lax.fori_loop(0, _NCLS // 8, max_body, (vm0, vi0))

                def sum_body(cc, s):
                    for k in range(8):
                        c = cc * 8 + k
                        cl = t_v[0, a, 5 + c, sl]
                        s = s + jnp.exp(cl - vm)
                    return s

                s = lax.fori_loop(0, _NCLS // 8, sum_body,
                                  jnp.full((_L,), 0.0, jnp.float32))
                conf = 1.0 / s
                cid = vi.astype(jnp.float32)

                for f, val in enumerate([xs, ys, ws, hs, det, conf, cid]):
                    ob_v[0, 0, f, sl] = val
                kb_v[0, 0, sl] = jnp.where(det > th, 1.0, 0.0)
            pltpu.sync_copy(
                ob_v,
                boxes_hbm.at[pl.ds(b, 1), pl.ds(a, 1), :, pl.ds(j0, _CH)])
            pltpu.sync_copy(
                kb_v, keep_hbm.at[pl.ds(b, 1), pl.ds(a, 1), pl.ds(j0, _CH)])
        return carry

    lax.fori_loop(0, nchunks // nw, chunk_body, 0)


def _sc_decode(x, th, b_off, ksc, *, A, C, H, W, aw, ah):
    n = H * W
    info = plsc.get_sparse_core_info()
    nw = info.num_cores * info.num_subcores
    mesh = plsc.VectorSubcoreMesh(core_axis_name="c", subcore_axis_name="s")
    body = functools.partial(_sc_body, A=A, H=H, W=W, aw=aw, ah=ah,
                             nw=nw, b_off=b_off, ksc=ksc)
    return pl.kernel(
        body,
        mesh=mesh,
        out_type=[
            jax.ShapeDtypeStruct((ksc, A, 7, n), jnp.float32),
            jax.ShapeDtypeStruct((ksc, A, n), jnp.float32),
        ],
        scratch_types=[
            pltpu.VMEM((1, A, C // A, _CH), jnp.float32),
            pltpu.VMEM((1, 1, 7, _CH), jnp.float32),
            pltpu.VMEM((1, 1, _CH), jnp.float32),
            pltpu.VMEM((_L,), jnp.float32),
        ],
    )(x, jnp.broadcast_to(th, (_L,)))


def kernel(output, nms_tresh):
    B, C, H, W = output.shape
    A = len(_ANCHOR_MASK)
    n = H * W
    x = output.reshape(B, A, C // A, n)
    th = jnp.asarray(nms_tresh, jnp.float32)

    aw = tuple(_ANCHORS[m * 2] / _STRIDE / W for m in _ANCHOR_MASK)
    ah = tuple(_ANCHORS[m * 2 + 1] / _STRIDE / H for m in _ANCHOR_MASK)
    dims = dict(A=A, C=C, H=H, W=W, aw=aw, ah=ah)

    cpb_ok = n % _CH == 0
    ksc = _KSC if (cpb_ok and _KSC < B and (_KSC * (n // _CH)) % 32 == 0) else 0
    btc = B - ksc

    if ksc:
        sc_boxes, sc_keepf = _sc_decode(x, th, btc, ksc, **dims)
    tc_boxes, tc_keep = _tc_decode(x, th, btc, **dims)

    if ksc:
        boxes_t = jnp.concatenate([tc_boxes, sc_boxes], axis=0)
        keep = jnp.concatenate(
            [tc_keep.reshape(btc, A * n), sc_keepf.reshape(ksc, A * n) > 0.5],
            axis=0)
    else:
        boxes_t = tc_boxes
        keep = tc_keep.reshape(B, A * n)

    boxes = boxes_t.transpose(0, 1, 3, 2).reshape(B, A * n, 7)
    return boxes, keep


# speedup vs baseline: 1.3685x; 1.0964x over previous
import functools

import jax
import jax.numpy as jnp
from jax import lax
from jax.experimental import pallas as pl
from jax.experimental.pallas import tpu as pltpu
from jax.experimental.pallas import tpu_sc as plsc

_ANCHORS = [12.0, 16.0, 19.0, 36.0, 40.0, 28.0, 36.0, 75.0, 76.0, 55.0,
            72.0, 146.0, 142.0, 110.0, 192.0, 243.0, 459.0, 401.0]
_ANCHOR_MASK = [6, 7, 8]
_NCLS = 80
_STRIDE = 32
_BB = 8
_KSC = 8
_CH = 256
_L = 16


def _sigmoid(x):
    return 1.0 / (1.0 + jnp.exp(-x))


def _tc_body(thresh_ref, x_ref, boxes_ref, keep_ref, *, H, W, aw, ah, bb):
    n = H * W
    A = len(aw)
    idx = jax.lax.broadcasted_iota(jnp.int32, (1, n), 1)
    gx = (idx % W).astype(jnp.float32)
    gy = (idx // W).astype(jnp.float32)
    th = thresh_ref[0]

    for i in range(bb):
        for a in range(A):
            t = x_ref[i, a]
            xs = (_sigmoid(t[0:1, :]) + gx) * (1.0 / W)
            ys = (_sigmoid(t[1:2, :]) + gy) * (1.0 / H)
            ws = jnp.exp(t[2:3, :]) * aw[a]
            hs = jnp.exp(t[3:4, :]) * ah[a]
            det = _sigmoid(t[4:5, :])

            cls = t[5:5 + _NCLS, :]
            m = jnp.max(cls, axis=0, keepdims=True)
            s = jnp.sum(jnp.exp(cls - m), axis=0, keepdims=True)
            conf = 1.0 / s
            cidx = jax.lax.broadcasted_iota(jnp.int32, cls.shape, 0)
            first_max = jnp.min(
                jnp.where(cls == m, cidx, _NCLS), axis=0, keepdims=True)
            cid = first_max.astype(jnp.float32)

            boxes_ref[i, a] = jnp.concatenate(
                [xs, ys, ws, hs, det, conf, cid], axis=0)
            keep_ref[i, a] = det > th


def _tc_decode(x, th, Btc, *, A, C, H, W, aw, ah):
    n = H * W
    bb = _BB if Btc % _BB == 0 else 1
    body = functools.partial(_tc_body, H=H, W=W, aw=aw, ah=ah, bb=bb)
    return pl.pallas_call(
        body,
        grid=(Btc // bb,),
        in_specs=[
            pl.BlockSpec(memory_space=pltpu.SMEM),
            pl.BlockSpec((bb, A, C // A, n), lambda b: (b, 0, 0, 0)),
        ],
        out_specs=[
            pl.BlockSpec((bb, A, 7, n), lambda b: (b, 0, 0, 0)),
            pl.BlockSpec((bb, A, 1, n), lambda b: (b, 0, 0, 0)),
        ],
        out_shape=[
            jax.ShapeDtypeStruct((Btc, A, 7, n), jnp.float32),
            jax.ShapeDtypeStruct((Btc, A, 1, n), jnp.bool_),
        ],
    )(th.reshape(1), x)


def _sc_body(x_hbm, th_hbm, boxes_hbm, keep_hbm, t_v, ob_v, kb_v, th_v,
             *, A, H, W, aw, ah, nw, b_off, ksc):
    n = H * W
    cpb = n // _CH
    nchunks = ksc * cpb
    wid = lax.axis_index("s") * 2 + lax.axis_index("c")
    pltpu.sync_copy(th_hbm, th_v)
    th = th_v[:]
    iota = lax.iota(jnp.int32, _L)

    def chunk_body(it, carry):
        g = wid + it * nw
        b = g // cpb
        j0 = (g % cpb) * _CH
        pltpu.sync_copy(
            x_hbm.at[pl.ds(b_off + b, 1), :, :, pl.ds(j0, _CH)], t_v)
        for a in range(A):
            for v in range(_CH // _L):
                s16 = v * _L
                sl = pl.ds(s16, _L)
                ci = iota + (j0 + s16)
                gx = (ci & (W - 1)).astype(jnp.float32)
                gy = lax.shift_right_logical(
                    ci, W.bit_length() - 1).astype(jnp.float32)
                x0 = t_v[0, a, 0, sl]
                x1 = t_v[0, a, 1, sl]
                x2 = t_v[0, a, 2, sl]
                x3 = t_v[0, a, 3, sl]
                x4 = t_v[0, a, 4, sl]
                xs = (_sigmoid(x0) + gx) * (1.0 / W)
                ys = (_sigmoid(x1) + gy) * (1.0 / H)
                ws = jnp.exp(x2) * aw[a]
                hs = jnp.exp(x3) * ah[a]
                det = _sigmoid(x4)

                def max_body(cc, mc):
                    vm, vi = mc
                    for k in range(8):
                        c = cc * 8 + k
                        cl = t_v[0, a, 5 + c, sl]
                        gt = cl > vm
                        cvec = jnp.full((_L,), 1, jnp.int32) * c
                        vm = jnp.where(gt, cl, vm)
                        vi = jnp.where(gt, cvec, vi)
                    return vm, vi

                vm0 = jnp.full((_L,), -jnp.inf, jnp.float32)
                vi0 = jnp.full((_L,), 0, jnp.int32)
                vm, vi = lax.fori_loop(0, _NCLS // 8, max_body, (vm0, vi0))

                def sum_body(cc, s):
                    for k in range(8):
                        c = cc * 8 + k
                        cl = t_v[0, a, 5 + c, sl]
                        s = s + jnp.exp(cl - vm)
                    return s

                s = lax.fori_loop(0, _NCLS // 8, sum_body,
                                  jnp.full((_L,), 0.0, jnp.float32))
                conf = 1.0 / s
                cid = vi.astype(jnp.float32)

                for f, val in enumerate([xs, ys, ws, hs, det, conf, cid]):
                    ob_v[0, 0, f, sl] = val
                kb_v[0, 0, sl] = jnp.where(det > th, 1.0, 0.0)
            pltpu.sync_copy(
                ob_v,
                boxes_hbm.at[pl.ds(b, 1), pl.ds(a, 1), :, pl.ds(j0, _CH)])
            pltpu.sync_copy(
                kb_v, keep_hbm.at[pl.ds(b, 1), pl.ds(a, 1), pl.ds(j0, _CH)])
        return carry

    lax.fori_loop(0, nchunks // nw, chunk_body, 0)


def _sc_decode(x, th, b_off, ksc, *, A, C, H, W, aw, ah):
    n = H * W
    info = plsc.get_sparse_core_info()
    nw = info.num_cores * info.num_subcores
    mesh = plsc.VectorSubcoreMesh(core_axis_name="c", subcore_axis_name="s")
    body = functools.partial(_sc_body, A=A, H=H, W=W, aw=aw, ah=ah,
                             nw=nw, b_off=b_off, ksc=ksc)
    return pl.kernel(
        body,
        mesh=mesh,
        out_type=[
            jax.ShapeDtypeStruct((ksc, A, 7, n), jnp.float32),
            jax.ShapeDtypeStruct((ksc, A, n), jnp.float32),
        ],
        scratch_types=[
            pltpu.VMEM((1, A, C // A, _CH), jnp.float32),
            pltpu.VMEM((1, 1, 7, _CH), jnp.float32),
            pltpu.VMEM((1, 1, _CH), jnp.float32),
            pltpu.VMEM((_L,), jnp.float32),
        ],
    )(x, jnp.broadcast_to(th, (_L,)))


def kernel(output, nms_tresh):
    B, C, H, W = output.shape
    A = len(_ANCHOR_MASK)
    n = H * W
    x = output.reshape(B, A, C // A, n)
    th = jnp.asarray(nms_tresh, jnp.float32)

    aw = tuple(_ANCHORS[m * 2] / _STRIDE / W for m in _ANCHOR_MASK)
    ah = tuple(_ANCHORS[m * 2 + 1] / _STRIDE / H for m in _ANCHOR_MASK)
    dims = dict(A=A, C=C, H=H, W=W, aw=aw, ah=ah)

    cpb_ok = n % _CH == 0
    ksc = _KSC if (cpb_ok and _KSC < B and (_KSC * (n // _CH)) % 32 == 0) else 0
    btc = B - ksc

    if ksc:
        sc_boxes, sc_keepf = _sc_decode(x, th, btc, ksc, **dims)
    tc_boxes, tc_keep = _tc_decode(x, th, btc, **dims)

    if ksc:
        boxes_t = jnp.concatenate([tc_boxes, sc_boxes], axis=0)
        keep = jnp.concatenate(
            [tc_keep.reshape(btc, A * n), sc_keepf.reshape(ksc, A * n) > 0.5],
            axis=0)
    else:
        boxes_t = tc_boxes
        keep = tc_keep.reshape(B, A * n)

    boxes = boxes_t.transpose(0, 1, 3, 2).reshape(B, A * n, 7)
    return boxes, keep
